# Initial kernel scaffold; baseline (speedup 1.0000x reference)
import jax
import jax.numpy as jnp
from jax.experimental import pallas as pl


def kernel(association_pairs, drug_embedding, disease_embedding, W_drug, W_disease):
    raise NotImplementedError("write your pallas kernel here")


# speedup vs baseline: 2.7043x; 2.7043x over previous
import functools

import jax
import jax.numpy as jnp
from jax import lax
from jax.experimental import pallas as pl
from jax.experimental.pallas import tpu as pltpu
from jax.experimental.pallas import tpu_sc as plsc

LANES = 16
NC, NS = 2, 16
NW = NC * NS
CHUNK = 128


def _transpose_body(wd_ref, ws_ref, td_ref, ts_ref):
    td_ref[...] = wd_ref[...].T
    ts_ref[...] = ws_ref[...].T


def _transpose_tables(W_drug, W_disease):
    emb, nd = W_drug.shape
    _, ns = W_disease.shape
    return pl.pallas_call(
        _transpose_body,
        out_shape=(
            jax.ShapeDtypeStruct((nd, emb), jnp.float32),
            jax.ShapeDtypeStruct((ns, emb), jnp.float32),
        ),
    )(W_drug, W_disease)


def _lane_splat(vec, i):
    idx = jnp.full((LANES,), i, dtype=jnp.int32)
    return lax.gather(
        vec,
        idx[:, None],
        dimension_numbers=lax.GatherDimensionNumbers(
            offset_dims=(), collapsed_slice_dims=(0,), start_index_map=(0,)
        ),
        slice_sizes=(1,),
        mode=lax.GatherScatterMode.PROMISE_IN_BOUNDS,
    )


def _newton_rsqrt(t):
    i = lax.bitcast_convert_type(t, jnp.int32)
    y = lax.bitcast_convert_type(jnp.int32(0x5F3759DF) - (i >> 1), jnp.float32)
    for _ in range(3):
        y = y * (1.5 - 0.5 * t * y * y)
    return y


def _make_sc_interact(B, EMB):
    BPW = B // NW
    NCHUNK = BPW // CHUNK
    NBLK = BPW // LANES
    JV = EMB // LANES

    mesh = plsc.VectorSubcoreMesh(core_axis_name="c", subcore_axis_name="s")

    @functools.partial(
        pl.kernel,
        mesh=mesh,
        compiler_params=pltpu.CompilerParams(
            needs_layout_passes=False, use_tc_tiling_on_sc=False),
        out_type=jax.ShapeDtypeStruct((B, EMB), jnp.float32),
        scratch_types=[
            pltpu.VMEM((NCHUNK, CHUNK), jnp.int32),
            pltpu.VMEM((NCHUNK, CHUNK), jnp.int32),
            pltpu.VMEM((BPW, EMB), jnp.float32),
            pltpu.VMEM((BPW, EMB), jnp.float32),
            pltpu.VMEM((LANES * LANES,), jnp.float32),
            pltpu.SemaphoreType.DMA,
            pltpu.SemaphoreType.DMA,
        ],
    )
    def sc_interact(a0_hbm, a1_hbm, tabd_hbm, tabs_hbm, out_hbm,
                    idx0_v, idx1_v, r0_v, r1_v, sqT_v, sem0, sem1):
        wid = lax.axis_index("s") * NC + lax.axis_index("c")
        base = wid * BPW

        pltpu.sync_copy(a0_hbm.at[wid], idx0_v)
        pltpu.sync_copy(a1_hbm.at[wid], idx1_v)

        waits = []
        for j in range(NCHUNK):
            dst = pl.ds(j * CHUNK, CHUNK)
            waits.append(
                pltpu.async_copy(tabd_hbm.at[idx0_v.at[j]], r0_v.at[dst], sem0))
            waits.append(
                pltpu.async_copy(tabs_hbm.at[idx1_v.at[j]], r1_v.at[dst], sem1))
        for w in waits:
            w.wait()

        iota = lax.iota(jnp.int32, LANES)

        def block(blk, _):
            r = blk * LANES
            for i in range(LANES):
                row = r + i
                s = jnp.zeros((LANES,), jnp.float32)
                for j in range(JV):
                    dd = pl.ds(j * LANES, LANES)
                    p = r0_v[row, dd] * r1_v[row, dd]
                    r0_v[row, dd] = p
                    s = s + p * p
                plsc.store_scatter(sqT_v, [iota * LANES + i], s)
            acc = sqT_v[pl.ds(0, LANES)]
            for l in range(1, LANES):
                acc = acc + sqT_v[pl.ds(l * LANES, LANES)]
            y = _newton_rsqrt(jnp.maximum(acc, 1e-35))
            inv = 1.0 / jnp.maximum(acc * y, 1e-12)
            for i in range(LANES):
                row = r + i
                g = _lane_splat(inv, i)
                for j in range(JV):
                    dd = pl.ds(j * LANES, LANES)
                    r0_v[row, dd] = r0_v[row, dd] * g
            return _

        lax.fori_loop(0, NBLK, block, None)

        pltpu.sync_copy(r0_v, out_hbm.at[pl.ds(base, BPW)])

    return sc_interact


def kernel(association_pairs, drug_embedding, disease_embedding, W_drug,
           W_disease):
    del drug_embedding, disease_embedding
    B = association_pairs.shape[1]
    EMB = W_drug.shape[0]
    tabd, tabs = _transpose_tables(W_drug, W_disease)
    a0 = association_pairs[0].reshape(NW, B // NW // CHUNK, CHUNK)
    a1 = association_pairs[1].reshape(NW, B // NW // CHUNK, CHUNK)
    return _make_sc_interact(B, EMB)(a0, a1, tabd, tabs)
